# baseline (speedup 1.0000x reference)
import functools

import jax
import jax.numpy as jnp
from jax import lax
from jax.experimental import pallas as pl
from jax.experimental.pallas import tpu as pltpu
from jax.experimental.pallas import tpu_sc as plsc

_NC = 2
_NS = 16
_NW = _NC * _NS


def _silu(x):
    return x * jax.nn.sigmoid(x)


def _proj_body(x_ref, ws_ref, wr_ref, ps_ref, pr_ref):
    x = x_ref[...]
    ps_ref[...] = jnp.dot(x, ws_ref[...], preferred_element_type=jnp.float32)
    pr_ref[...] = jnp.dot(x, wr_ref[...], preferred_element_type=jnp.float32)


def _node_projections(x_nodes, w1s, w1r):
    n, d = x_nodes.shape
    return pl.pallas_call(
        _proj_body,
        out_shape=[jax.ShapeDtypeStruct((n, d), jnp.float32)] * 2,
    )(x_nodes, w1s, w1r)


def _sc_gather_body(e_per_w, ch, d, ps_hbm, pr_hbm, snd_hbm, rcv_hbm, g_hbm,
                    isall, irall, bs0, br0, bs1, br1,
                    sgs0, sgr0, sgs1, sgr1, sw0, sw1):
    wid = lax.axis_index("s") * _NC + lax.axis_index("c")
    base = wid * e_per_w
    n_ch = e_per_w // ch
    bs = (bs0, bs1)
    br = (br0, br1)
    sgs = (sgs0, sgs1)
    sgr = (sgr0, sgr1)
    sw = (sw0, sw1)

    pltpu.sync_copy(snd_hbm.at[pl.ds(base, e_per_w)], isall)
    pltpu.sync_copy(rcv_hbm.at[pl.ds(base, e_per_w)], irall)

    def issue_gather(i, s):
        off = i * ch
        pltpu.async_copy(ps_hbm.at[isall.at[pl.ds(off, ch)]], bs[s], sgs[s])
        pltpu.async_copy(pr_hbm.at[irall.at[pl.ds(off, ch)]], br[s], sgr[s])

    def stage(i, s, o):
        @pl.when(i >= 1)
        def _():
            pltpu.make_async_copy(bs[o], g_hbm.at[pl.ds(base, ch)],
                                  sw[o]).wait()

        @pl.when(i + 1 < n_ch)
        def _():
            issue_gather(i + 1, o)

        pltpu.make_async_copy(ps_hbm.at[isall.at[pl.ds(0, ch)]], bs[s],
                              sgs[s]).wait()
        pltpu.make_async_copy(pr_hbm.at[irall.at[pl.ds(0, ch)]], br[s],
                              sgr[s]).wait()

        def row(j, carry):
            for cc in range(d // 16):
                sl = pl.ds(cc * 16, 16)
                plsc.addupdate(bs[s].at[j, sl], br[s][j, sl])
            return carry

        lax.fori_loop(0, ch, row, 0)
        pltpu.async_copy(bs[s], g_hbm.at[pl.ds(base + i * ch, ch)], sw[s])

    issue_gather(0, 0)

    def pair(k, carry):
        stage(2 * k, 0, 1)

        @pl.when(2 * k + 1 < n_ch)
        def _():
            stage(2 * k + 1, 1, 0)

        return carry

    lax.fori_loop(0, (n_ch + 1) // 2, pair, 0)
    last = n_ch - 1
    pltpu.make_async_copy(bs[last % 2], g_hbm.at[pl.ds(base, ch)],
                          sw[last % 2]).wait()


def _sc_gather_add(psend, precv, sender, receiver):
    n, d = psend.shape
    e = sender.shape[0]
    e_per_w = e // _NW
    ch = 80
    mesh = plsc.VectorSubcoreMesh(core_axis_name="c", subcore_axis_name="s")
    k = functools.partial(
        pl.kernel,
        out_type=jax.ShapeDtypeStruct((e, d), jnp.float32),
        mesh=mesh,
        scratch_types=[
            pltpu.VMEM((e_per_w,), jnp.int32),
            pltpu.VMEM((e_per_w,), jnp.int32),
            pltpu.VMEM((ch, d), jnp.float32),
            pltpu.VMEM((ch, d), jnp.float32),
            pltpu.VMEM((ch, d), jnp.float32),
            pltpu.VMEM((ch, d), jnp.float32),
            pltpu.SemaphoreType.DMA,
            pltpu.SemaphoreType.DMA,
            pltpu.SemaphoreType.DMA,
            pltpu.SemaphoreType.DMA,
            pltpu.SemaphoreType.DMA,
            pltpu.SemaphoreType.DMA,
        ],
        compiler_params=pltpu.CompilerParams(needs_layout_passes=False),
    )(functools.partial(_sc_gather_body, e_per_w, ch, d))
    return k(psend, precv, sender, receiver)


def _edge_body(g_ref, xe_ref, w1x_ref, b1e_ref, w2e_ref, b2e_ref, out_ref,
               outt_ref):
    xp = jnp.dot(xe_ref[...], w1x_ref[...], preferred_element_type=jnp.float32)
    pre = g_ref[...] + xp + b1e_ref[...]
    h = _silu(pre).astype(jnp.bfloat16)
    w2e = w2e_ref[...].astype(jnp.bfloat16)
    pre2 = jnp.dot(h, w2e, preferred_element_type=jnp.float32)
    out_ref[...] = _silu(pre2 + b2e_ref[...])
    pre2t = lax.dot_general(w2e, h, (((0,), (1,)), ((), ())),
                            preferred_element_type=jnp.float32)
    outt_ref[...] = _silu(pre2t + b2e_ref[...].reshape(-1, 1))


def _edge_mlp(g, x_edges, w1x, b1e, w2e, b2e):
    e, d = g.shape
    de = x_edges.shape[1]
    be = 2560
    grid = e // be
    return pl.pallas_call(
        _edge_body,
        grid=(grid,),
        in_specs=[
            pl.BlockSpec((be, d), lambda i: (i, 0)),
            pl.BlockSpec((be, de), lambda i: (i, 0)),
            pl.BlockSpec((de, d), lambda i: (0, 0)),
            pl.BlockSpec((1, d), lambda i: (0, 0)),
            pl.BlockSpec((d, de), lambda i: (0, 0)),
            pl.BlockSpec((1, de), lambda i: (0, 0)),
        ],
        out_specs=[
            pl.BlockSpec((be, de), lambda i: (i, 0)),
            pl.BlockSpec((de, be), lambda i: (0, i)),
        ],
        out_shape=[
            jax.ShapeDtypeStruct((e, de), jnp.float32),
            jax.ShapeDtypeStruct((de, e), jnp.float32),
        ],
    )(g, x_edges, w1x, b1e.reshape(1, d), w2e, b2e.reshape(1, de))


def _sc_scatter_body(n, e_per_c, ch, net_hbm, rcv_hbm, out_hbm,
                     idx0, val0, idx1, val1, acc, si0, sv0, si1, sv1):
    cid = lax.axis_index("c")
    sid = lax.axis_index("s")
    e = e_per_c * _NC
    idxb = (idx0, idx1)
    valb = (val0, val1)
    si = (si0, si1)
    sv = (sv0, sv1)
    ebase = cid * e_per_c
    tbase = sid * e + ebase
    n_ch = e_per_c // ch

    def issue_load(i, s):
        pltpu.async_copy(rcv_hbm.at[pl.ds(ebase + i * ch, ch)], idxb[s], si[s])
        pltpu.async_copy(net_hbm.at[pl.ds(tbase + i * ch, ch)], valb[s], sv[s])

    issue_load(0, 0)

    def zrow(j, carry):
        acc[pl.ds(j * 16, 16)] = jnp.zeros((16,), jnp.float32)
        return carry

    lax.fori_loop(0, n // 16, zrow, 0)

    def stage(i, s, o):
        @pl.when(i + 1 < n_ch)
        def _():
            issue_load(i + 1, o)

        pltpu.make_async_copy(rcv_hbm.at[pl.ds(ebase, ch)], idxb[s],
                              si[s]).wait()
        pltpu.make_async_copy(net_hbm.at[pl.ds(tbase, ch)], valb[s],
                              sv[s]).wait()

        def grp(j, carry):
            for g in range(5):
                sl = pl.ds(j * 80 + g * 16, 16)
                plsc.addupdate_scatter(acc, [idxb[s][sl]], valb[s][sl])
            return carry

        lax.fori_loop(0, ch // 80, grp, 0)

    def pair(k, carry):
        stage(2 * k, 0, 1)
        stage(2 * k + 1, 1, 0)
        return carry

    lax.fori_loop(0, n_ch // 2, pair, 0)
    wid = cid * _NS + sid
    pltpu.sync_copy(acc, out_hbm.at[pl.ds(wid * n, n)])


def _sc_segment_sum(new_edges_t_flat, receiver, n, de, ch):
    e = receiver.shape[0]
    e_per_c = e // _NC
    mesh = plsc.VectorSubcoreMesh(core_axis_name="c", subcore_axis_name="s")
    k = functools.partial(
        pl.kernel,
        out_type=jax.ShapeDtypeStruct((_NC * _NS * n,), jnp.float32),
        mesh=mesh,
        scratch_types=[
            pltpu.VMEM((ch,), jnp.int32),
            pltpu.VMEM((ch,), jnp.float32),
            pltpu.VMEM((ch,), jnp.int32),
            pltpu.VMEM((ch,), jnp.float32),
            pltpu.VMEM((n,), jnp.float32),
            pltpu.SemaphoreType.DMA,
            pltpu.SemaphoreType.DMA,
            pltpu.SemaphoreType.DMA,
            pltpu.SemaphoreType.DMA,
        ],
        compiler_params=pltpu.CompilerParams(needs_layout_passes=False),
    )(functools.partial(_sc_scatter_body, n, e_per_c, ch))
    return k(new_edges_t_flat, receiver).reshape(_NC, _NS, n)


def _node_body(x_ref, p0_ref, p1_ref, p2_ref, p3_ref, w1x_ref, w1a_ref,
               b1n_ref, w2n_ref, b2n_ref, out_ref):
    aggr_t = (p0_ref[...] + p1_ref[...]) + (p2_ref[...] + p3_ref[...])
    acontrib = lax.dot_general(aggr_t, w1a_ref[...], (((0,), (0,)), ((), ())),
                               preferred_element_type=jnp.float32)
    pre = (jnp.dot(x_ref[...], w1x_ref[...], preferred_element_type=jnp.float32)
           + acontrib + b1n_ref[...])
    hn = _silu(pre)
    out_ref[...] = (jnp.dot(hn, w2n_ref[...], preferred_element_type=jnp.float32)
                    + b2n_ref[...])


def _node_mlp(x_nodes, partials, w1nx, w1na, b1n, w2n, b2n):
    n, d = x_nodes.shape
    p0, p1, p2, p3 = partials
    return pl.pallas_call(
        _node_body,
        out_shape=jax.ShapeDtypeStruct((n, d), jnp.float32),
    )(x_nodes, p0, p1, p2, p3, w1nx, w1na, b1n.reshape(1, d), w2n,
      b2n.reshape(1, d))


def kernel(x_nodes, x_edges, edge_index, W1e, b1e, W2e, b2e, W1n, b1n, W2n,
           b2n):
    n, d = x_nodes.shape
    de = x_edges.shape[1]
    sender = edge_index[0]
    receiver = edge_index[1]

    w1s = W1e[:d]
    w1r = W1e[d:2 * d]
    w1x = W1e[2 * d:]

    e = sender.shape[0]
    e1 = (e * 3 // 5 // (80 * _NW)) * (80 * _NW)
    psend, precv = _node_projections(x_nodes, w1s, w1r)
    g_a = _sc_gather_add(psend, precv, sender[:e1], receiver[:e1])
    g_b = _sc_gather_add(psend, precv, sender[e1:], receiver[e1:])
    ne_a, net_a = _edge_mlp(g_a, x_edges[:e1], w1x, b1e, W2e, b2e)
    parts_a = _sc_segment_sum(net_a.reshape(-1), receiver[:e1], n, de, 8000)
    ne_b, net_b = _edge_mlp(g_b, x_edges[e1:], w1x, b1e, W2e, b2e)
    parts_b = _sc_segment_sum(net_b.reshape(-1), receiver[e1:], n, de, 8000)
    new_edges = jnp.concatenate([ne_a, ne_b], axis=0)
    new_nodes = _node_mlp(x_nodes, (parts_a[0], parts_a[1], parts_b[0],
                                    parts_b[1]), W1n[:d], W1n[d:],
                          b1n, W2n, b2n)
    return new_nodes, new_edges


# speedup vs baseline: 1.0399x; 1.0399x over previous
import functools

import jax
import jax.numpy as jnp
from jax import lax
from jax.experimental import pallas as pl
from jax.experimental.pallas import tpu as pltpu
from jax.experimental.pallas import tpu_sc as plsc

_NC = 2
_NS = 16
_NW = _NC * _NS


def _silu(x):
    return x * jax.nn.sigmoid(x)


def _proj_body(x_ref, ws_ref, wr_ref, ps_ref, pr_ref):
    x = x_ref[...]
    ps_ref[...] = jnp.dot(x, ws_ref[...], preferred_element_type=jnp.float32)
    pr_ref[...] = jnp.dot(x, wr_ref[...], preferred_element_type=jnp.float32)


def _node_projections(x_nodes, w1s, w1r):
    n, d = x_nodes.shape
    return pl.pallas_call(
        _proj_body,
        out_shape=[jax.ShapeDtypeStruct((n, d), jnp.float32)] * 2,
    )(x_nodes, w1s, w1r)


def _sc_gather_body(e_per_w, ch, d, ps_hbm, pr_hbm, snd_hbm, rcv_hbm, g_hbm,
                    isall, irall, bs0, br0, bs1, br1,
                    sgs0, sgr0, sgs1, sgr1, sw0, sw1):
    wid = lax.axis_index("s") * _NC + lax.axis_index("c")
    base = wid * e_per_w
    n_ch = e_per_w // ch
    bs = (bs0, bs1)
    br = (br0, br1)
    sgs = (sgs0, sgs1)
    sgr = (sgr0, sgr1)
    sw = (sw0, sw1)

    pltpu.sync_copy(snd_hbm.at[pl.ds(base, e_per_w)], isall)
    pltpu.sync_copy(rcv_hbm.at[pl.ds(base, e_per_w)], irall)

    def issue_gather(i, s):
        off = i * ch
        pltpu.async_copy(ps_hbm.at[isall.at[pl.ds(off, ch)]], bs[s], sgs[s])
        pltpu.async_copy(pr_hbm.at[irall.at[pl.ds(off, ch)]], br[s], sgr[s])

    def stage(i, s, o):
        @pl.when(i >= 1)
        def _():
            pltpu.make_async_copy(bs[o], g_hbm.at[pl.ds(base, ch)],
                                  sw[o]).wait()

        @pl.when(i + 1 < n_ch)
        def _():
            issue_gather(i + 1, o)

        pltpu.make_async_copy(ps_hbm.at[isall.at[pl.ds(0, ch)]], bs[s],
                              sgs[s]).wait()
        pltpu.make_async_copy(pr_hbm.at[irall.at[pl.ds(0, ch)]], br[s],
                              sgr[s]).wait()

        def row(j, carry):
            for cc in range(d // 16):
                sl = pl.ds(cc * 16, 16)
                plsc.addupdate(bs[s].at[j, sl], br[s][j, sl])
            return carry

        lax.fori_loop(0, ch, row, 0)
        pltpu.async_copy(bs[s], g_hbm.at[pl.ds(base + i * ch, ch)], sw[s])

    issue_gather(0, 0)

    def pair(k, carry):
        stage(2 * k, 0, 1)

        @pl.when(2 * k + 1 < n_ch)
        def _():
            stage(2 * k + 1, 1, 0)

        return carry

    lax.fori_loop(0, (n_ch + 1) // 2, pair, 0)
    last = n_ch - 1
    pltpu.make_async_copy(bs[last % 2], g_hbm.at[pl.ds(base, ch)],
                          sw[last % 2]).wait()


def _sc_gather_add(psend, precv, sender, receiver):
    n, d = psend.shape
    e = sender.shape[0]
    e_per_w = e // _NW
    ch = 80
    mesh = plsc.VectorSubcoreMesh(core_axis_name="c", subcore_axis_name="s")
    k = functools.partial(
        pl.kernel,
        out_type=jax.ShapeDtypeStruct((e, d), jnp.float32),
        mesh=mesh,
        scratch_types=[
            pltpu.VMEM((e_per_w,), jnp.int32),
            pltpu.VMEM((e_per_w,), jnp.int32),
            pltpu.VMEM((ch, d), jnp.float32),
            pltpu.VMEM((ch, d), jnp.float32),
            pltpu.VMEM((ch, d), jnp.float32),
            pltpu.VMEM((ch, d), jnp.float32),
            pltpu.SemaphoreType.DMA,
            pltpu.SemaphoreType.DMA,
            pltpu.SemaphoreType.DMA,
            pltpu.SemaphoreType.DMA,
            pltpu.SemaphoreType.DMA,
            pltpu.SemaphoreType.DMA,
        ],
        compiler_params=pltpu.CompilerParams(needs_layout_passes=False),
    )(functools.partial(_sc_gather_body, e_per_w, ch, d))
    return k(psend, precv, sender, receiver)


def _edge_body(g_ref, xe_ref, w1x_ref, b1e_ref, w2e_ref, b2e_ref, out_ref,
               outt_ref):
    xp = jnp.dot(xe_ref[...], w1x_ref[...], preferred_element_type=jnp.float32)
    pre = g_ref[...] + xp + b1e_ref[...]
    h = _silu(pre).astype(jnp.bfloat16)
    w2e = w2e_ref[...].astype(jnp.bfloat16)
    pre2 = jnp.dot(h, w2e, preferred_element_type=jnp.float32)
    out_ref[...] = _silu(pre2 + b2e_ref[...])
    pre2t = lax.dot_general(w2e, h, (((0,), (1,)), ((), ())),
                            preferred_element_type=jnp.float32)
    outt_ref[...] = _silu(pre2t + b2e_ref[...].reshape(-1, 1))


def _edge_mlp(g, x_edges, w1x, b1e, w2e, b2e):
    e, d = g.shape
    de = x_edges.shape[1]
    be = 2560
    grid = e // be
    return pl.pallas_call(
        _edge_body,
        grid=(grid,),
        in_specs=[
            pl.BlockSpec((be, d), lambda i: (i, 0)),
            pl.BlockSpec((be, de), lambda i: (i, 0)),
            pl.BlockSpec((de, d), lambda i: (0, 0)),
            pl.BlockSpec((1, d), lambda i: (0, 0)),
            pl.BlockSpec((d, de), lambda i: (0, 0)),
            pl.BlockSpec((1, de), lambda i: (0, 0)),
        ],
        out_specs=[
            pl.BlockSpec((be, de), lambda i: (i, 0)),
            pl.BlockSpec((de, be), lambda i: (0, i)),
        ],
        out_shape=[
            jax.ShapeDtypeStruct((e, de), jnp.float32),
            jax.ShapeDtypeStruct((de, e), jnp.float32),
        ],
    )(g, x_edges, w1x, b1e.reshape(1, d), w2e, b2e.reshape(1, de))


def _sc_scatter_body(n, e_per_c, ch, net_hbm, rcv_hbm, out_hbm,
                     idx0, val0, idx1, val1, acc, si0, sv0, si1, sv1):
    cid = lax.axis_index("c")
    sid = lax.axis_index("s")
    e = e_per_c * _NC
    idxb = (idx0, idx1)
    valb = (val0, val1)
    si = (si0, si1)
    sv = (sv0, sv1)
    ebase = cid * e_per_c
    tbase = sid * e + ebase
    n_ch = e_per_c // ch

    def issue_load(i, s):
        pltpu.async_copy(rcv_hbm.at[pl.ds(ebase + i * ch, ch)], idxb[s], si[s])
        pltpu.async_copy(net_hbm.at[pl.ds(tbase + i * ch, ch)], valb[s], sv[s])

    issue_load(0, 0)

    def zrow(j, carry):
        acc[pl.ds(j * 16, 16)] = jnp.zeros((16,), jnp.float32)
        return carry

    lax.fori_loop(0, n // 16, zrow, 0)

    def stage(i, s, o):
        @pl.when(i + 1 < n_ch)
        def _():
            issue_load(i + 1, o)

        pltpu.make_async_copy(rcv_hbm.at[pl.ds(ebase, ch)], idxb[s],
                              si[s]).wait()
        pltpu.make_async_copy(net_hbm.at[pl.ds(tbase, ch)], valb[s],
                              sv[s]).wait()

        def grp(j, carry):
            for g in range(5):
                sl = pl.ds(j * 80 + g * 16, 16)
                plsc.addupdate_scatter(acc, [idxb[s][sl]], valb[s][sl])
            return carry

        lax.fori_loop(0, ch // 80, grp, 0)

    def pair(k, carry):
        stage(2 * k, 0, 1)
        stage(2 * k + 1, 1, 0)
        return carry

    lax.fori_loop(0, n_ch // 2, pair, 0)
    wid = cid * _NS + sid
    pltpu.sync_copy(acc, out_hbm.at[pl.ds(wid * n, n)])


def _sc_segment_sum(new_edges_t_flat, receiver, n, de, ch):
    e = receiver.shape[0]
    e_per_c = e // _NC
    mesh = plsc.VectorSubcoreMesh(core_axis_name="c", subcore_axis_name="s")
    k = functools.partial(
        pl.kernel,
        out_type=jax.ShapeDtypeStruct((_NC * _NS * n,), jnp.float32),
        mesh=mesh,
        scratch_types=[
            pltpu.VMEM((ch,), jnp.int32),
            pltpu.VMEM((ch,), jnp.float32),
            pltpu.VMEM((ch,), jnp.int32),
            pltpu.VMEM((ch,), jnp.float32),
            pltpu.VMEM((n,), jnp.float32),
            pltpu.SemaphoreType.DMA,
            pltpu.SemaphoreType.DMA,
            pltpu.SemaphoreType.DMA,
            pltpu.SemaphoreType.DMA,
        ],
        compiler_params=pltpu.CompilerParams(needs_layout_passes=False),
    )(functools.partial(_sc_scatter_body, n, e_per_c, ch))
    return k(new_edges_t_flat, receiver).reshape(_NC, _NS, n)


def _node_body(x_ref, p0_ref, p1_ref, w1x_ref, w1a_ref,
               b1n_ref, w2n_ref, b2n_ref, out_ref):
    aggr_t = p0_ref[...] + p1_ref[...]
    acontrib = lax.dot_general(aggr_t, w1a_ref[...], (((0,), (0,)), ((), ())),
                               preferred_element_type=jnp.float32)
    pre = (jnp.dot(x_ref[...], w1x_ref[...], preferred_element_type=jnp.float32)
           + acontrib + b1n_ref[...])
    hn = _silu(pre)
    out_ref[...] = (jnp.dot(hn, w2n_ref[...], preferred_element_type=jnp.float32)
                    + b2n_ref[...])


def _node_mlp(x_nodes, partials, w1nx, w1na, b1n, w2n, b2n):
    n, d = x_nodes.shape
    p0, p1 = partials
    return pl.pallas_call(
        _node_body,
        out_shape=jax.ShapeDtypeStruct((n, d), jnp.float32),
    )(x_nodes, p0, p1, w1nx, w1na, b1n.reshape(1, d), w2n,
      b2n.reshape(1, d))


def kernel(x_nodes, x_edges, edge_index, W1e, b1e, W2e, b2e, W1n, b1n, W2n,
           b2n):
    n, d = x_nodes.shape
    de = x_edges.shape[1]
    sender = edge_index[0]
    receiver = edge_index[1]

    w1s = W1e[:d]
    w1r = W1e[d:2 * d]
    w1x = W1e[2 * d:]

    psend, precv = _node_projections(x_nodes, w1s, w1r)
    g = _sc_gather_add(psend, precv, sender, receiver)
    new_edges, new_edges_t = _edge_mlp(g, x_edges, w1x, b1e, W2e, b2e)
    parts = _sc_segment_sum(new_edges_t.reshape(-1), receiver, n, de, 10000)
    new_nodes = _node_mlp(x_nodes, (parts[0], parts[1]),
                          W1n[:d], W1n[d:], b1n, W2n, b2n)
    return new_nodes, new_edges


# speedup vs baseline: 1.1130x; 1.0703x over previous
import functools

import jax
import jax.numpy as jnp
from jax import lax
from jax.experimental import pallas as pl
from jax.experimental.pallas import tpu as pltpu
from jax.experimental.pallas import tpu_sc as plsc

_NC = 2
_NS = 16
_NW = _NC * _NS


def _silu(x):
    return x * jax.nn.sigmoid(x)


def _proj_body(x_ref, ws_ref, wr_ref, ps_ref, pr_ref):
    x = x_ref[...]
    ps_ref[...] = jnp.dot(x, ws_ref[...], preferred_element_type=jnp.float32)
    pr_ref[...] = jnp.dot(x, wr_ref[...], preferred_element_type=jnp.float32)


def _node_projections(x_nodes, w1s, w1r):
    n, d = x_nodes.shape
    return pl.pallas_call(
        _proj_body,
        out_shape=[jax.ShapeDtypeStruct((n, d), jnp.float32)] * 2,
    )(x_nodes, w1s, w1r)


def _sc_gather_body(e_per_w, ch, d, ps_hbm, pr_hbm, snd_hbm, rcv_hbm, g_hbm,
                    isall, irall, bs0, br0, bs1, br1,
                    sgs0, sgr0, sgs1, sgr1, sw0, sw1):
    wid = lax.axis_index("s") * _NC + lax.axis_index("c")
    base = wid * e_per_w
    n_ch = e_per_w // ch
    bs = (bs0, bs1)
    br = (br0, br1)
    sgs = (sgs0, sgs1)
    sgr = (sgr0, sgr1)
    sw = (sw0, sw1)

    pltpu.sync_copy(snd_hbm.at[pl.ds(base, e_per_w)], isall)
    pltpu.sync_copy(rcv_hbm.at[pl.ds(base, e_per_w)], irall)

    def issue_gather(i, s):
        off = i * ch
        pltpu.async_copy(ps_hbm.at[isall.at[pl.ds(off, ch)]], bs[s], sgs[s])
        pltpu.async_copy(pr_hbm.at[irall.at[pl.ds(off, ch)]], br[s], sgr[s])

    def stage(i, s, o):
        @pl.when(i >= 1)
        def _():
            pltpu.make_async_copy(bs[o], g_hbm.at[pl.ds(base, ch)],
                                  sw[o]).wait()

        @pl.when(i + 1 < n_ch)
        def _():
            issue_gather(i + 1, o)

        pltpu.make_async_copy(ps_hbm.at[isall.at[pl.ds(0, ch)]], bs[s],
                              sgs[s]).wait()
        pltpu.make_async_copy(pr_hbm.at[irall.at[pl.ds(0, ch)]], br[s],
                              sgr[s]).wait()

        def row(j, carry):
            for cc in range(d // 16):
                sl = pl.ds(cc * 16, 16)
                plsc.addupdate(bs[s].at[j, sl], br[s][j, sl])
            return carry

        lax.fori_loop(0, ch, row, 0)
        pltpu.async_copy(bs[s], g_hbm.at[pl.ds(base + i * ch, ch)], sw[s])

    issue_gather(0, 0)

    def pair(k, carry):
        stage(2 * k, 0, 1)

        @pl.when(2 * k + 1 < n_ch)
        def _():
            stage(2 * k + 1, 1, 0)

        return carry

    lax.fori_loop(0, (n_ch + 1) // 2, pair, 0)
    last = n_ch - 1
    pltpu.make_async_copy(bs[last % 2], g_hbm.at[pl.ds(base, ch)],
                          sw[last % 2]).wait()


def _sc_gather_add(psend, precv, sender, receiver):
    n, d = psend.shape
    e = sender.shape[0]
    e_per_w = e // _NW
    ch = 80
    mesh = plsc.VectorSubcoreMesh(core_axis_name="c", subcore_axis_name="s")
    k = functools.partial(
        pl.kernel,
        out_type=jax.ShapeDtypeStruct((e, d), jnp.float32),
        mesh=mesh,
        scratch_types=[
            pltpu.VMEM((e_per_w,), jnp.int32),
            pltpu.VMEM((e_per_w,), jnp.int32),
            pltpu.VMEM((ch, d), jnp.float32),
            pltpu.VMEM((ch, d), jnp.float32),
            pltpu.VMEM((ch, d), jnp.float32),
            pltpu.VMEM((ch, d), jnp.float32),
            pltpu.SemaphoreType.DMA,
            pltpu.SemaphoreType.DMA,
            pltpu.SemaphoreType.DMA,
            pltpu.SemaphoreType.DMA,
            pltpu.SemaphoreType.DMA,
            pltpu.SemaphoreType.DMA,
        ],
        compiler_params=pltpu.CompilerParams(needs_layout_passes=False),
    )(functools.partial(_sc_gather_body, e_per_w, ch, d))
    return k(psend, precv, sender, receiver)


def _edge_body(g_ref, xe_ref, w1x_ref, b1e_ref, w2e_ref, b2e_ref, out_ref,
               outt_ref):
    xp = jnp.dot(xe_ref[...], w1x_ref[...], preferred_element_type=jnp.float32)
    pre = g_ref[...] + xp + b1e_ref[...]
    h = _silu(pre).astype(jnp.bfloat16)
    w2e = w2e_ref[...].astype(jnp.bfloat16)
    pre2 = jnp.dot(h, w2e, preferred_element_type=jnp.float32)
    out_ref[...] = _silu(pre2 + b2e_ref[...])
    pre2t = lax.dot_general(w2e, h, (((0,), (1,)), ((), ())),
                            preferred_element_type=jnp.float32)
    outt_ref[...] = _silu(pre2t + b2e_ref[...].reshape(-1, 1))


def _edge_mlp(g, x_edges, w1x, b1e, w2e, b2e):
    e, d = g.shape
    de = x_edges.shape[1]
    be = 6400
    grid = e // be
    return pl.pallas_call(
        _edge_body,
        grid=(grid,),
        in_specs=[
            pl.BlockSpec((be, d), lambda i: (i, 0)),
            pl.BlockSpec((be, de), lambda i: (i, 0)),
            pl.BlockSpec((de, d), lambda i: (0, 0)),
            pl.BlockSpec((1, d), lambda i: (0, 0)),
            pl.BlockSpec((d, de), lambda i: (0, 0)),
            pl.BlockSpec((1, de), lambda i: (0, 0)),
        ],
        out_specs=[
            pl.BlockSpec((be, de), lambda i: (i, 0)),
            pl.BlockSpec((de, be), lambda i: (0, i)),
        ],
        out_shape=[
            jax.ShapeDtypeStruct((e, de), jnp.float32),
            jax.ShapeDtypeStruct((de, e), jnp.float32),
        ],
    )(g, x_edges, w1x, b1e.reshape(1, d), w2e, b2e.reshape(1, de))


def _sc_scatter_body(n, e_per_c, ch, net_hbm, rcv_hbm, out_hbm,
                     idx0, val0, idx1, val1, acc, si0, sv0, si1, sv1):
    cid = lax.axis_index("c")
    sid = lax.axis_index("s")
    e = e_per_c * _NC
    idxb = (idx0, idx1)
    valb = (val0, val1)
    si = (si0, si1)
    sv = (sv0, sv1)
    ebase = cid * e_per_c
    tbase = sid * e + ebase
    n_ch = e_per_c // ch

    def issue_load(i, s):
        pltpu.async_copy(rcv_hbm.at[pl.ds(ebase + i * ch, ch)], idxb[s], si[s])
        pltpu.async_copy(net_hbm.at[pl.ds(tbase + i * ch, ch)], valb[s], sv[s])

    issue_load(0, 0)

    def zrow(j, carry):
        acc[pl.ds(j * 16, 16)] = jnp.zeros((16,), jnp.float32)
        return carry

    lax.fori_loop(0, n // 16, zrow, 0)

    def stage(i, s, o):
        @pl.when(i + 1 < n_ch)
        def _():
            issue_load(i + 1, o)

        pltpu.make_async_copy(rcv_hbm.at[pl.ds(ebase, ch)], idxb[s],
                              si[s]).wait()
        pltpu.make_async_copy(net_hbm.at[pl.ds(tbase, ch)], valb[s],
                              sv[s]).wait()

        def grp(j, carry):
            for g in range(5):
                sl = pl.ds(j * 80 + g * 16, 16)
                plsc.addupdate_scatter(acc, [idxb[s][sl]], valb[s][sl])
            return carry

        lax.fori_loop(0, ch // 80, grp, 0)

    def pair(k, carry):
        stage(2 * k, 0, 1)
        stage(2 * k + 1, 1, 0)
        return carry

    lax.fori_loop(0, n_ch // 2, pair, 0)
    wid = cid * _NS + sid
    pltpu.sync_copy(acc, out_hbm.at[pl.ds(wid * n, n)])


def _sc_segment_sum(new_edges_t_flat, receiver, n, de, ch):
    e = receiver.shape[0]
    e_per_c = e // _NC
    mesh = plsc.VectorSubcoreMesh(core_axis_name="c", subcore_axis_name="s")
    k = functools.partial(
        pl.kernel,
        out_type=jax.ShapeDtypeStruct((_NC * _NS * n,), jnp.float32),
        mesh=mesh,
        scratch_types=[
            pltpu.VMEM((ch,), jnp.int32),
            pltpu.VMEM((ch,), jnp.float32),
            pltpu.VMEM((ch,), jnp.int32),
            pltpu.VMEM((ch,), jnp.float32),
            pltpu.VMEM((n,), jnp.float32),
            pltpu.SemaphoreType.DMA,
            pltpu.SemaphoreType.DMA,
            pltpu.SemaphoreType.DMA,
            pltpu.SemaphoreType.DMA,
        ],
        compiler_params=pltpu.CompilerParams(needs_layout_passes=False),
    )(functools.partial(_sc_scatter_body, n, e_per_c, ch))
    return k(new_edges_t_flat, receiver).reshape(_NC, _NS, n)


def _node_body(x_ref, p0_ref, p1_ref, w1x_ref, w1a_ref,
               b1n_ref, w2n_ref, b2n_ref, out_ref):
    aggr_t = p0_ref[...] + p1_ref[...]
    acontrib = lax.dot_general(aggr_t, w1a_ref[...], (((0,), (0,)), ((), ())),
                               preferred_element_type=jnp.float32)
    pre = (jnp.dot(x_ref[...], w1x_ref[...], preferred_element_type=jnp.float32)
           + acontrib + b1n_ref[...])
    hn = _silu(pre)
    out_ref[...] = (jnp.dot(hn, w2n_ref[...], preferred_element_type=jnp.float32)
                    + b2n_ref[...])


def _node_mlp(x_nodes, partials, w1nx, w1na, b1n, w2n, b2n):
    n, d = x_nodes.shape
    p0, p1 = partials
    return pl.pallas_call(
        _node_body,
        out_shape=jax.ShapeDtypeStruct((n, d), jnp.float32),
    )(x_nodes, p0, p1, w1nx, w1na, b1n.reshape(1, d), w2n,
      b2n.reshape(1, d))


def kernel(x_nodes, x_edges, edge_index, W1e, b1e, W2e, b2e, W1n, b1n, W2n,
           b2n):
    n, d = x_nodes.shape
    de = x_edges.shape[1]
    sender = edge_index[0]
    receiver = edge_index[1]

    w1s = W1e[:d]
    w1r = W1e[d:2 * d]
    w1x = W1e[2 * d:]

    psend, precv = _node_projections(x_nodes, w1s, w1r)
    g = _sc_gather_add(psend, precv, sender, receiver)
    new_edges, new_edges_t = _edge_mlp(g, x_edges, w1x, b1e, W2e, b2e)
    parts = _sc_segment_sum(new_edges_t.reshape(-1), receiver, n, de, 10000)
    new_nodes = _node_mlp(x_nodes, (parts[0], parts[1]),
                          W1n[:d], W1n[d:], b1n, W2n, b2n)
    return new_nodes, new_edges


# speedup vs baseline: 1.1232x; 1.0091x over previous
import functools

import jax
import jax.numpy as jnp
from jax import lax
from jax.experimental import pallas as pl
from jax.experimental.pallas import tpu as pltpu
from jax.experimental.pallas import tpu_sc as plsc

_NC = 2
_NS = 16
_NW = _NC * _NS


def _silu(x):
    return x * jax.nn.sigmoid(x)


def _proj_body(x_ref, ws_ref, wr_ref, ps_ref, pr_ref):
    x = x_ref[...]
    ps_ref[...] = jnp.dot(x, ws_ref[...], preferred_element_type=jnp.float32)
    pr_ref[...] = jnp.dot(x, wr_ref[...], preferred_element_type=jnp.float32)


def _node_projections(x_nodes, w1s, w1r):
    n, d = x_nodes.shape
    return pl.pallas_call(
        _proj_body,
        out_shape=[jax.ShapeDtypeStruct((n, d), jnp.float32)] * 2,
    )(x_nodes, w1s, w1r)


def _sc_gather_body(e_per_w, ch, d, ps_hbm, pr_hbm, snd_hbm, rcv_hbm, g_hbm,
                    isall, irall, bs0, br0, bs1, br1,
                    sgs0, sgr0, sgs1, sgr1, sw0, sw1):
    wid = lax.axis_index("s") * _NC + lax.axis_index("c")
    base = wid * e_per_w
    n_ch = e_per_w // ch
    bs = (bs0, bs1)
    br = (br0, br1)
    sgs = (sgs0, sgs1)
    sgr = (sgr0, sgr1)
    sw = (sw0, sw1)

    pltpu.sync_copy(snd_hbm.at[pl.ds(base, e_per_w)], isall)
    pltpu.sync_copy(rcv_hbm.at[pl.ds(base, e_per_w)], irall)

    def issue_gather(i, s):
        off = i * ch
        pltpu.async_copy(ps_hbm.at[isall.at[pl.ds(off, ch)]], bs[s], sgs[s])
        pltpu.async_copy(pr_hbm.at[irall.at[pl.ds(off, ch)]], br[s], sgr[s])

    def stage(i, s, o):
        @pl.when(i >= 1)
        def _():
            pltpu.make_async_copy(bs[o], g_hbm.at[pl.ds(base, ch)],
                                  sw[o]).wait()

        @pl.when(i + 1 < n_ch)
        def _():
            issue_gather(i + 1, o)

        pltpu.make_async_copy(ps_hbm.at[isall.at[pl.ds(0, ch)]], bs[s],
                              sgs[s]).wait()
        pltpu.make_async_copy(pr_hbm.at[irall.at[pl.ds(0, ch)]], br[s],
                              sgr[s]).wait()

        def row(j, carry):
            for cc in range(d // 16):
                sl = pl.ds(cc * 16, 16)
                plsc.addupdate(bs[s].at[j, sl], br[s][j, sl])
            return carry

        lax.fori_loop(0, ch, row, 0)
        pltpu.async_copy(bs[s], g_hbm.at[pl.ds(base + i * ch, ch)], sw[s])

    issue_gather(0, 0)

    def pair(k, carry):
        stage(2 * k, 0, 1)

        @pl.when(2 * k + 1 < n_ch)
        def _():
            stage(2 * k + 1, 1, 0)

        return carry

    lax.fori_loop(0, (n_ch + 1) // 2, pair, 0)
    last = n_ch - 1
    pltpu.make_async_copy(bs[last % 2], g_hbm.at[pl.ds(base, ch)],
                          sw[last % 2]).wait()


def _sc_gather_add(psend, precv, sender, receiver):
    n, d = psend.shape
    e = sender.shape[0]
    e_per_w = e // _NW
    ch = 80
    mesh = plsc.VectorSubcoreMesh(core_axis_name="c", subcore_axis_name="s")
    k = functools.partial(
        pl.kernel,
        out_type=jax.ShapeDtypeStruct((e, d), jnp.float32),
        mesh=mesh,
        scratch_types=[
            pltpu.VMEM((e_per_w,), jnp.int32),
            pltpu.VMEM((e_per_w,), jnp.int32),
            pltpu.VMEM((ch, d), jnp.float32),
            pltpu.VMEM((ch, d), jnp.float32),
            pltpu.VMEM((ch, d), jnp.float32),
            pltpu.VMEM((ch, d), jnp.float32),
            pltpu.SemaphoreType.DMA,
            pltpu.SemaphoreType.DMA,
            pltpu.SemaphoreType.DMA,
            pltpu.SemaphoreType.DMA,
            pltpu.SemaphoreType.DMA,
            pltpu.SemaphoreType.DMA,
        ],
        compiler_params=pltpu.CompilerParams(needs_layout_passes=False),
    )(functools.partial(_sc_gather_body, e_per_w, ch, d))
    return k(psend, precv, sender, receiver)


def _edge_body(g_ref, xe_ref, w1x_ref, b1e_ref, w2e_ref, b2e_ref, out_ref,
               outt_ref):
    xp = jnp.dot(xe_ref[...], w1x_ref[...], preferred_element_type=jnp.float32)
    pre = g_ref[...] + xp + b1e_ref[...]
    h = _silu(pre).astype(jnp.bfloat16)
    w2e = w2e_ref[...].astype(jnp.bfloat16)
    pre2 = jnp.dot(h, w2e, preferred_element_type=jnp.float32)
    out_ref[...] = _silu(pre2 + b2e_ref[...])
    pre2t = lax.dot_general(w2e, h, (((0,), (1,)), ((), ())),
                            preferred_element_type=jnp.float32)
    outt_ref[...] = _silu(pre2t + b2e_ref[...].reshape(-1, 1))


def _edge_mlp(g, x_edges, w1x, b1e, w2e, b2e):
    e, d = g.shape
    de = x_edges.shape[1]
    be = 12800
    grid = e // be
    return pl.pallas_call(
        _edge_body,
        grid=(grid,),
        in_specs=[
            pl.BlockSpec((be, d), lambda i: (i, 0)),
            pl.BlockSpec((be, de), lambda i: (i, 0)),
            pl.BlockSpec((de, d), lambda i: (0, 0)),
            pl.BlockSpec((1, d), lambda i: (0, 0)),
            pl.BlockSpec((d, de), lambda i: (0, 0)),
            pl.BlockSpec((1, de), lambda i: (0, 0)),
        ],
        out_specs=[
            pl.BlockSpec((be, de), lambda i: (i, 0)),
            pl.BlockSpec((de, be), lambda i: (0, i)),
        ],
        out_shape=[
            jax.ShapeDtypeStruct((e, de), jnp.float32),
            jax.ShapeDtypeStruct((de, e), jnp.float32),
        ],
    )(g, x_edges, w1x, b1e.reshape(1, d), w2e, b2e.reshape(1, de))


def _sc_scatter_body(n, e_per_c, ch, net_hbm, rcv_hbm, out_hbm,
                     idx0, val0, idx1, val1, acc, si0, sv0, si1, sv1):
    cid = lax.axis_index("c")
    sid = lax.axis_index("s")
    e = e_per_c * _NC
    idxb = (idx0, idx1)
    valb = (val0, val1)
    si = (si0, si1)
    sv = (sv0, sv1)
    ebase = cid * e_per_c
    tbase = sid * e + ebase
    n_ch = e_per_c // ch

    def issue_load(i, s):
        pltpu.async_copy(rcv_hbm.at[pl.ds(ebase + i * ch, ch)], idxb[s], si[s])
        pltpu.async_copy(net_hbm.at[pl.ds(tbase + i * ch, ch)], valb[s], sv[s])

    issue_load(0, 0)

    def zrow(j, carry):
        acc[pl.ds(j * 16, 16)] = jnp.zeros((16,), jnp.float32)
        return carry

    lax.fori_loop(0, n // 16, zrow, 0)

    def stage(i, s, o):
        @pl.when(i + 1 < n_ch)
        def _():
            issue_load(i + 1, o)

        pltpu.make_async_copy(rcv_hbm.at[pl.ds(ebase, ch)], idxb[s],
                              si[s]).wait()
        pltpu.make_async_copy(net_hbm.at[pl.ds(tbase, ch)], valb[s],
                              sv[s]).wait()

        def grp(j, carry):
            for g in range(5):
                sl = pl.ds(j * 80 + g * 16, 16)
                plsc.addupdate_scatter(acc, [idxb[s][sl]], valb[s][sl])
            return carry

        lax.fori_loop(0, ch // 80, grp, 0)

    def pair(k, carry):
        stage(2 * k, 0, 1)
        stage(2 * k + 1, 1, 0)
        return carry

    lax.fori_loop(0, n_ch // 2, pair, 0)
    wid = cid * _NS + sid
    pltpu.sync_copy(acc, out_hbm.at[pl.ds(wid * n, n)])


def _sc_segment_sum(new_edges_t_flat, receiver, n, de, ch):
    e = receiver.shape[0]
    e_per_c = e // _NC
    mesh = plsc.VectorSubcoreMesh(core_axis_name="c", subcore_axis_name="s")
    k = functools.partial(
        pl.kernel,
        out_type=jax.ShapeDtypeStruct((_NC * _NS * n,), jnp.float32),
        mesh=mesh,
        scratch_types=[
            pltpu.VMEM((ch,), jnp.int32),
            pltpu.VMEM((ch,), jnp.float32),
            pltpu.VMEM((ch,), jnp.int32),
            pltpu.VMEM((ch,), jnp.float32),
            pltpu.VMEM((n,), jnp.float32),
            pltpu.SemaphoreType.DMA,
            pltpu.SemaphoreType.DMA,
            pltpu.SemaphoreType.DMA,
            pltpu.SemaphoreType.DMA,
        ],
        compiler_params=pltpu.CompilerParams(needs_layout_passes=False),
    )(functools.partial(_sc_scatter_body, n, e_per_c, ch))
    return k(new_edges_t_flat, receiver).reshape(_NC, _NS, n)


def _node_body(x_ref, p0_ref, p1_ref, w1x_ref, w1a_ref,
               b1n_ref, w2n_ref, b2n_ref, out_ref):
    aggr_t = p0_ref[...] + p1_ref[...]
    acontrib = lax.dot_general(aggr_t, w1a_ref[...], (((0,), (0,)), ((), ())),
                               preferred_element_type=jnp.float32)
    pre = (jnp.dot(x_ref[...], w1x_ref[...], preferred_element_type=jnp.float32)
           + acontrib + b1n_ref[...])
    hn = _silu(pre)
    out_ref[...] = (jnp.dot(hn, w2n_ref[...], preferred_element_type=jnp.float32)
                    + b2n_ref[...])


def _node_mlp(x_nodes, partials, w1nx, w1na, b1n, w2n, b2n):
    n, d = x_nodes.shape
    p0, p1 = partials
    return pl.pallas_call(
        _node_body,
        out_shape=jax.ShapeDtypeStruct((n, d), jnp.float32),
    )(x_nodes, p0, p1, w1nx, w1na, b1n.reshape(1, d), w2n,
      b2n.reshape(1, d))


def kernel(x_nodes, x_edges, edge_index, W1e, b1e, W2e, b2e, W1n, b1n, W2n,
           b2n):
    n, d = x_nodes.shape
    de = x_edges.shape[1]
    sender = edge_index[0]
    receiver = edge_index[1]

    w1s = W1e[:d]
    w1r = W1e[d:2 * d]
    w1x = W1e[2 * d:]

    psend, precv = _node_projections(x_nodes, w1s, w1r)
    g = _sc_gather_add(psend, precv, sender, receiver)
    new_edges, new_edges_t = _edge_mlp(g, x_edges, w1x, b1e, W2e, b2e)
    parts = _sc_segment_sum(new_edges_t.reshape(-1), receiver, n, de, 10000)
    new_nodes = _node_mlp(x_nodes, (parts[0], parts[1]),
                          W1n[:d], W1n[d:], b1n, W2n, b2n)
    return new_nodes, new_edges
